# Initial kernel scaffold; baseline (speedup 1.0000x reference)
import jax
import jax.numpy as jnp
from jax.experimental import pallas as pl


def kernel(t, xh, node_mask, edge_mask, edge_attributes, params):
    raise NotImplementedError("write your pallas kernel here")


# speedup vs baseline: 10.4149x; 10.4149x over previous
import jax
import jax.numpy as jnp
from jax.experimental import pallas as pl
from jax.experimental.pallas import tpu as pltpu

HID = 64
N_LAYERS = 4
INV_SUB = 2
N = 48
E = N * N
NORM = 100.0


def _silu(v):
    return v * jax.nn.sigmoid(v)


def _egnn_kernel(t_ref, xh_ref, nm_ref, em_ref, ea_ref,
                 embw_ref, embb_ref,
                 w1r_ref, w1c_ref, w1e_ref, b1_ref, w2_ref, b2_ref,
                 attw_ref, attb_ref,
                 n1h_ref, n1a_ref, bn1_ref, n2_ref, bn2_ref,
                 c3_ref, outw_ref, outb_ref,
                 hf_ref, vel_ref):
    nm = nm_ref[0]
    em = em_ref[0]
    ea = ea_ref[0]
    tval = t_ref[0, 0, 0]

    xh_m = xh_ref[0] * nm
    h_raw = xh_m[:, :8]
    x0 = xh_m[:, 8:11]

    h = (jnp.dot(h_raw, embw_ref[:8, :], preferred_element_type=jnp.float32)
         + tval * embw_ref[8:9, :] + embb_ref[0:1, :])

    def rep_dst(v):
        c = v.shape[1]
        return jax.lax.broadcast_in_dim(v, (N, N, c), (0, 2)).reshape(E, c)

    def rep_src(v):
        c = v.shape[1]
        return jax.lax.broadcast_in_dim(v, (N, N, c), (1, 2)).reshape(E, c)

    ii = jax.lax.broadcasted_iota(jnp.int32, (N, E), 0)
    ee = jax.lax.broadcasted_iota(jnp.int32, (N, E), 1)
    A = jnp.where(ee // N == ii, 1.0, 0.0).astype(jnp.float32)

    x = x0
    dist_cat = None
    for layer in range(N_LAYERS):
        xi = rep_dst(x)
        xj = rep_src(x)
        cd = xi - xj
        d2 = jnp.sum(cd * cd, axis=1, keepdims=True)
        cd = cd / jnp.sqrt(d2 + 1e-8)
        if layer == 0:
            dist_cat = jnp.concatenate([d2, ea], axis=1)
        edge_attr = jnp.concatenate([d2, dist_cat], axis=1)

        for g in range(INV_SUB):
            mi = layer * 3 + g
            ga = layer * INV_SUB + g
            hr = jnp.dot(h, w1r_ref[mi], preferred_element_type=jnp.float32)
            hc = jnp.dot(h, w1c_ref[mi], preferred_element_type=jnp.float32)
            et = jnp.dot(edge_attr, w1e_ref[mi],
                         preferred_element_type=jnp.float32)
            m = _silu(rep_dst(hr) + rep_src(hc) + et + b1_ref[mi])
            m = _silu(jnp.dot(m, w2_ref[mi],
                              preferred_element_type=jnp.float32) + b2_ref[mi])
            att = jax.nn.sigmoid(
                jnp.sum(m * attw_ref[ga:ga + 1, :], axis=1, keepdims=True)
                + attb_ref[ga:ga + 1, 0:1])
            ef = m * att * em
            agg = jnp.dot(A, ef, preferred_element_type=jnp.float32) / NORM
            nmlp = _silu(
                jnp.dot(h, n1h_ref[ga], preferred_element_type=jnp.float32)
                + jnp.dot(agg, n1a_ref[ga], preferred_element_type=jnp.float32)
                + bn1_ref[ga])
            nmlp = (jnp.dot(nmlp, n2_ref[ga],
                            preferred_element_type=jnp.float32) + bn2_ref[ga])
            h = (h + nmlp) * nm

        mi = layer * 3 + 2
        hr = jnp.dot(h, w1r_ref[mi], preferred_element_type=jnp.float32)
        hc = jnp.dot(h, w1c_ref[mi], preferred_element_type=jnp.float32)
        et = jnp.dot(edge_attr, w1e_ref[mi], preferred_element_type=jnp.float32)
        it = _silu(rep_dst(hr) + rep_src(hc) + et + b1_ref[mi])
        it = _silu(jnp.dot(it, w2_ref[mi],
                           preferred_element_type=jnp.float32) + b2_ref[mi])
        phi = jnp.sum(it * c3_ref[layer:layer + 1, :], axis=1, keepdims=True)
        trans = cd * phi * em
        aggx = jnp.dot(A, trans, preferred_element_type=jnp.float32) / NORM
        x = (x + aggx) * nm
        h = h * nm

    hf = (jnp.dot(h, outw_ref[...], preferred_element_type=jnp.float32)
          + outb_ref[0:1, :]) * nm
    vel = (x - x0) * nm
    ncnt = jnp.sum(nm)
    mean = jnp.sum(vel, axis=0, keepdims=True) / ncnt
    vel = vel - mean * nm

    hf_ref[0] = hf
    vel_ref[0] = vel


def kernel(t, xh, node_mask, edge_mask, edge_attributes, params):
    bs, n, dims = xh.shape
    p = params

    w1r, w1c, w1e, b1, w2, b2 = [], [], [], [], [], []
    attw, attb, n1h, n1a, bn1, n2, bn2, c3 = [], [], [], [], [], [], [], []
    for b in range(N_LAYERS):
        for g in range(INV_SUB):
            pre = 'b%d_g%d_' % (b, g)
            w1 = p[pre + 'e1_w']
            w1r.append(w1[:HID]); w1c.append(w1[HID:2 * HID]); w1e.append(w1[2 * HID:])
            b1.append(p[pre + 'e1_b'])
            w2.append(p[pre + 'e2_w']); b2.append(p[pre + 'e2_b'])
            attw.append(p[pre + 'att_w'][:, 0]); attb.append(p[pre + 'att_b'])
            wn1 = p[pre + 'n1_w']
            n1h.append(wn1[:HID]); n1a.append(wn1[HID:])
            bn1.append(p[pre + 'n1_b'])
            n2.append(p[pre + 'n2_w']); bn2.append(p[pre + 'n2_b'])
        pre = 'b%d_eq_' % b
        w1 = p[pre + 'c1_w']
        w1r.append(w1[:HID]); w1c.append(w1[HID:2 * HID]); w1e.append(w1[2 * HID:])
        b1.append(p[pre + 'c1_b'])
        w2.append(p[pre + 'c2_w']); b2.append(p[pre + 'c2_b'])
        c3.append(p[pre + 'c3_w'][:, 0])

    W1R = jnp.stack(w1r); W1C = jnp.stack(w1c); W1E = jnp.stack(w1e)
    B1 = jnp.stack(b1)[:, None, :]
    W2 = jnp.stack(w2); B2 = jnp.stack(b2)[:, None, :]
    ATTW = jnp.stack(attw)
    ATTB = jnp.stack(attb)
    N1H = jnp.stack(n1h); N1A = jnp.stack(n1a)
    BN1 = jnp.stack(bn1)[:, None, :]
    N2 = jnp.stack(n2); BN2 = jnp.stack(bn2)[:, None, :]
    C3 = jnp.stack(c3)
    EMBW = p['emb_w']
    EMBB = p['emb_b'][None, :]
    OUTW = p['out_w']
    OUTB = p['out_b'][None, :]

    t2 = t.reshape(bs, 1, 1)
    nm3 = node_mask.reshape(bs, n, 1)
    em3 = edge_mask.reshape(bs, n * n, 1)
    ea3 = edge_attributes.reshape(bs, n * n, 2)

    def full(a):
        nd = a.ndim
        return pl.BlockSpec(a.shape, lambda b, _n=nd: (0,) * _n)

    grid = (bs,)
    in_specs = [
        pl.BlockSpec((1, 1, 1), lambda b: (b, 0, 0)),
        pl.BlockSpec((1, n, dims), lambda b: (b, 0, 0)),
        pl.BlockSpec((1, n, 1), lambda b: (b, 0, 0)),
        pl.BlockSpec((1, n * n, 1), lambda b: (b, 0, 0)),
        pl.BlockSpec((1, n * n, 2), lambda b: (b, 0, 0)),
    ] + [full(a) for a in (EMBW, EMBB, W1R, W1C, W1E, B1, W2, B2,
                           ATTW, ATTB, N1H, N1A, BN1, N2, BN2,
                           C3, OUTW, OUTB)]
    out_specs = (
        pl.BlockSpec((1, n, 3), lambda b: (b, 0, 0)),
        pl.BlockSpec((1, n, 3), lambda b: (b, 0, 0)),
    )
    out_shape = (
        jax.ShapeDtypeStruct((bs, n, 3), jnp.float32),
        jax.ShapeDtypeStruct((bs, n, 3), jnp.float32),
    )

    hf, vel = pl.pallas_call(
        _egnn_kernel,
        grid=grid,
        in_specs=in_specs,
        out_specs=out_specs,
        out_shape=out_shape,
        compiler_params=pltpu.CompilerParams(
            dimension_semantics=("arbitrary",)),
    )(t2, xh, nm3, em3, ea3,
      EMBW, EMBB, W1R, W1C, W1E, B1, W2, B2,
      ATTW, ATTB, N1H, N1A, BN1, N2, BN2, C3, OUTW, OUTB)

    return hf.reshape(bs * n, 3), vel


# speedup vs baseline: 29.1225x; 2.7962x over previous
import numpy as np
import jax
import jax.numpy as jnp
from jax.experimental import pallas as pl
from jax.experimental.pallas import tpu as pltpu

HID = 64
N_LAYERS = 4
INV_SUB = 2
N = 48
E = N * N
NORM = 100.0


def _silu(v):
    return v * jax.nn.sigmoid(v)


def _egnn_kernel(t_ref, xhh_ref, xhx_ref, nm_ref, em_ref, ea_ref,
                 embw_ref, embwt_ref, embb_ref,
                 w1r_ref, w1c_ref, w1e_ref, b1_ref, w2_ref, b2_ref,
                 attw_ref, attb_ref,
                 n1h_ref, n1a_ref, bn1_ref, n2_ref, bn2_ref,
                 c3_ref, outw_ref, outb_ref,
                 a_ref, sel128_ref, sel16_ref, sel6_ref, seg2_ref,
                 seg6_ref, qd2_ref, qd0_ref, pea_ref,
                 hf_ref, vel_ref):
    nm2 = nm_ref[0]
    em2 = em_ref[0]
    ea4 = ea_ref[0]
    t2 = t_ref[0]
    A = a_ref[...]

    def mm(a, b):
        return jnp.dot(a, b, preferred_element_type=jnp.float32)

    nm128 = mm(nm2, sel128_ref[...])
    nm16 = mm(nm2, sel16_ref[...])
    nm6 = mm(nm2, sel6_ref[...])

    h_raw = xhh_ref[0] * nm16
    x0 = xhx_ref[0] * nm6

    h = (mm(h_raw, embw_ref[...])
         + mm(t2, sel128_ref[...]) * embwt_ref[...]
         + embb_ref[...])

    def rep_dst(v):
        c = v.shape[1]
        return jax.lax.broadcast_in_dim(v, (N, N, c), (0, 2)).reshape(E, c)

    def rep_src(v):
        c = v.shape[1]
        return jax.lax.broadcast_in_dim(v, (N, N, c), (1, 2)).reshape(E, c)

    x = x0
    ea_static = None
    for layer in range(N_LAYERS):
        xi = rep_dst(x)
        xj = rep_src(x)
        cd = xi - xj
        sq = cd * cd
        d2_6 = mm(sq, seg6_ref[...])
        cd = cd / jnp.sqrt(d2_6 + 1e-8)
        if layer == 0:
            ea_static = mm(sq, qd0_ref[...]) + mm(ea4, pea_ref[...])
        edge8 = mm(sq, qd2_ref[...]) + ea_static

        for g in range(INV_SUB):
            mi = layer * 3 + g
            ga = layer * INV_SUB + g
            hr = mm(h, w1r_ref[mi])
            hc = mm(h, w1c_ref[mi])
            et = mm(edge8, w1e_ref[mi])
            m = _silu(rep_dst(hr) + rep_src(hc) + et + b1_ref[mi])
            m = _silu(mm(m, w2_ref[mi]) + b2_ref[mi])
            att2 = jax.nn.sigmoid(mm(m * attw_ref[ga], seg2_ref[...])
                                  + attb_ref[ga])
            ef = m * mm(att2 * em2, sel128_ref[...])
            agg = mm(A, ef) / NORM
            nmlp = _silu(mm(h, n1h_ref[ga]) + mm(agg, n1a_ref[ga])
                         + bn1_ref[ga])
            nmlp = mm(nmlp, n2_ref[ga]) + bn2_ref[ga]
            h = (h + nmlp) * nm128

        mi = layer * 3 + 2
        hr = mm(h, w1r_ref[mi])
        hc = mm(h, w1c_ref[mi])
        et = mm(edge8, w1e_ref[mi])
        it = _silu(rep_dst(hr) + rep_src(hc) + et + b1_ref[mi])
        it = _silu(mm(it, w2_ref[mi]) + b2_ref[mi])
        phi2 = mm(it * c3_ref[layer], seg2_ref[...])
        trans = cd * mm(phi2 * em2, sel6_ref[...])
        aggx = mm(A, trans) / NORM
        x = (x + aggx) * nm6
        h = h * nm128

    hf = (mm(h, outw_ref[...]) + outb_ref[...]) * nm6
    vel = (x - x0) * nm6
    ncnt6 = mm(jnp.sum(nm2, axis=0, keepdims=True), sel6_ref[...])
    mean = jnp.sum(vel, axis=0, keepdims=True) / ncnt6
    vel = vel - mean * nm6

    hf_ref[0] = hf
    vel_ref[0] = vel


def _block_diag2(w):
    s, a, b = w.shape
    z = jnp.zeros((s, 2 * a, 2 * b), jnp.float32)
    return z.at[:, :a, :b].set(w).at[:, a:, b:].set(w)


def kernel(t, xh, node_mask, edge_mask, edge_attributes, params):
    bs, n, dims = xh.shape
    bs2 = bs // 2
    p = params

    w1r, w1c, w1e, b1, w2, b2 = [], [], [], [], [], []
    attw, attb, n1h, n1a, bn1, n2, bn2, c3 = [], [], [], [], [], [], [], []
    for b in range(N_LAYERS):
        for g in range(INV_SUB):
            pre = 'b%d_g%d_' % (b, g)
            w1 = p[pre + 'e1_w']
            w1r.append(w1[:HID]); w1c.append(w1[HID:2 * HID]); w1e.append(w1[2 * HID:])
            b1.append(p[pre + 'e1_b'])
            w2.append(p[pre + 'e2_w']); b2.append(p[pre + 'e2_b'])
            attw.append(p[pre + 'att_w'][:, 0]); attb.append(p[pre + 'att_b'])
            wn1 = p[pre + 'n1_w']
            n1h.append(wn1[:HID]); n1a.append(wn1[HID:])
            bn1.append(p[pre + 'n1_b'])
            n2.append(p[pre + 'n2_w']); bn2.append(p[pre + 'n2_b'])
        pre = 'b%d_eq_' % b
        w1 = p[pre + 'c1_w']
        w1r.append(w1[:HID]); w1c.append(w1[HID:2 * HID]); w1e.append(w1[2 * HID:])
        b1.append(p[pre + 'c1_b'])
        w2.append(p[pre + 'c2_w']); b2.append(p[pre + 'c2_b'])
        c3.append(p[pre + 'c3_w'][:, 0])

    W1R = _block_diag2(jnp.stack(w1r))
    W1C = _block_diag2(jnp.stack(w1c))
    W1E = _block_diag2(jnp.stack(w1e))
    W2 = _block_diag2(jnp.stack(w2))
    N1H = _block_diag2(jnp.stack(n1h))
    N1A = _block_diag2(jnp.stack(n1a))
    N2 = _block_diag2(jnp.stack(n2))
    B1 = jnp.tile(jnp.stack(b1), (1, 2))[:, None, :]
    B2 = jnp.tile(jnp.stack(b2), (1, 2))[:, None, :]
    BN1 = jnp.tile(jnp.stack(bn1), (1, 2))[:, None, :]
    BN2 = jnp.tile(jnp.stack(bn2), (1, 2))[:, None, :]
    ATTW = jnp.tile(jnp.stack(attw), (1, 2))[:, None, :]
    ATTB = jnp.tile(jnp.stack(attb), (1, 2))[:, None, :]
    C3 = jnp.tile(jnp.stack(c3), (1, 2))[:, None, :]
    EMBW = _block_diag2(p['emb_w'][None, :8, :])[0]
    EMBWT = jnp.tile(p['emb_w'][8:9, :], (1, 2))
    EMBB = jnp.tile(p['emb_b'][None, :], (1, 2))
    OUTW = _block_diag2(p['out_w'][None])[0]
    OUTB = jnp.tile(p['out_b'][None, :], (1, 2))

    lane = np.arange(128)
    SEL128 = (lane[None, :] // HID == np.arange(2)[:, None]).astype(np.float32)
    SEL16 = (np.arange(16)[None, :] // 8 == np.arange(2)[:, None]).astype(np.float32)
    SEL6 = (np.arange(6)[None, :] // 3 == np.arange(2)[:, None]).astype(np.float32)
    SEG2 = SEL128.T.copy()
    SEG6 = (np.arange(6)[:, None] // 3 == np.arange(6)[None, :] // 3).astype(np.float32)
    K62 = SEL6.T
    PD2 = np.zeros((2, 8), np.float32); PD2[0, 0] = PD2[1, 4] = 1.0
    PD0 = np.zeros((2, 8), np.float32); PD0[0, 1] = PD0[1, 5] = 1.0
    QD2 = (K62 @ PD2).astype(np.float32)
    QD0 = (K62 @ PD0).astype(np.float32)
    PEA = np.zeros((4, 8), np.float32)
    PEA[0, 2] = PEA[1, 3] = PEA[2, 6] = PEA[3, 7] = 1.0
    AMAT = (np.arange(E)[None, :] // N == np.arange(N)[:, None]).astype(np.float32)

    consts = [jnp.asarray(v) for v in
              (AMAT, SEL128, SEL16, SEL6, SEG2, SEG6, QD2, QD0, PEA)]

    t2 = t.reshape(bs2, 1, 2)
    xh4 = xh.reshape(bs2, 2, n, dims).transpose(0, 2, 1, 3)
    xhh = xh4[:, :, :, :8].reshape(bs2, n, 16)
    xhx = xh4[:, :, :, 8:].reshape(bs2, n, 6)
    nm2 = node_mask.reshape(bs2, 2, n).transpose(0, 2, 1)
    em2 = edge_mask.reshape(bs2, 2, E).transpose(0, 2, 1)
    ea2 = (edge_attributes.reshape(bs2, 2, E, 2)
           .transpose(0, 2, 1, 3).reshape(bs2, E, 4))

    def full(a):
        nd = a.ndim
        return pl.BlockSpec(a.shape, lambda b, _n=nd: (0,) * _n)

    weights = (EMBW, EMBWT, EMBB, W1R, W1C, W1E, B1, W2, B2,
               ATTW, ATTB, N1H, N1A, BN1, N2, BN2, C3, OUTW, OUTB)
    in_specs = [
        pl.BlockSpec((1, 1, 2), lambda b: (b, 0, 0)),
        pl.BlockSpec((1, n, 16), lambda b: (b, 0, 0)),
        pl.BlockSpec((1, n, 6), lambda b: (b, 0, 0)),
        pl.BlockSpec((1, n, 2), lambda b: (b, 0, 0)),
        pl.BlockSpec((1, E, 2), lambda b: (b, 0, 0)),
        pl.BlockSpec((1, E, 4), lambda b: (b, 0, 0)),
    ] + [full(a) for a in weights] + [full(a) for a in consts]
    out_specs = (
        pl.BlockSpec((1, n, 6), lambda b: (b, 0, 0)),
        pl.BlockSpec((1, n, 6), lambda b: (b, 0, 0)),
    )
    out_shape = (
        jax.ShapeDtypeStruct((bs2, n, 6), jnp.float32),
        jax.ShapeDtypeStruct((bs2, n, 6), jnp.float32),
    )

    hf, vel = pl.pallas_call(
        _egnn_kernel,
        grid=(bs2,),
        in_specs=in_specs,
        out_specs=out_specs,
        out_shape=out_shape,
        compiler_params=pltpu.CompilerParams(
            dimension_semantics=("arbitrary",)),
    )(t2, xhh, xhx, nm2, em2, ea2, *weights, *consts)

    hf = hf.reshape(bs2, n, 2, 3).transpose(0, 2, 1, 3).reshape(bs * n, 3)
    vel = vel.reshape(bs2, n, 2, 3).transpose(0, 2, 1, 3).reshape(bs, n, 3)
    return hf, vel


# speedup vs baseline: 29.2276x; 1.0036x over previous
import numpy as np
import jax
import jax.numpy as jnp
from jax.experimental import pallas as pl
from jax.experimental.pallas import tpu as pltpu

HID = 64
N_LAYERS = 4
INV_SUB = 2
N = 48
E = N * N
NORM = 100.0


def _silu(v):
    return v * jax.nn.sigmoid(v)


def _egnn_kernel(t_ref, xhh_ref, xhx_ref, nm_ref, em_ref, ea_ref,
                 embw_ref, embwt_ref, embb_ref,
                 w1r_ref, w1c_ref, w1e_ref, b1_ref, w2_ref, b2_ref,
                 attw_ref, attb_ref,
                 n1h_ref, n1a_ref, bn1_ref, n2_ref, bn2_ref,
                 c3_ref, outw_ref, outb_ref,
                 a_ref, sel128_ref, sel16_ref, sel6_ref, seg2_ref,
                 seg6_ref, qd2_ref, qd0_ref, pea_ref,
                 hf_ref, vel_ref):
    nm2 = nm_ref[0]
    em2 = em_ref[0]
    ea4 = ea_ref[0]
    t2 = t_ref[0]
    A = a_ref[...]

    def mm(a, b):
        return jnp.dot(a, b, preferred_element_type=jnp.float32)

    nm128 = mm(nm2, sel128_ref[...])
    nm16 = mm(nm2, sel16_ref[...])
    nm6 = mm(nm2, sel6_ref[...])

    h_raw = xhh_ref[0] * nm16
    x0 = xhx_ref[0] * nm6

    h = (mm(h_raw, embw_ref[...])
         + mm(t2, sel128_ref[...]) * embwt_ref[...]
         + embb_ref[...])

    def rep_dst(v):
        c = v.shape[1]
        return jax.lax.broadcast_in_dim(v, (N, N, c), (0, 2)).reshape(E, c)

    def rep_src(v):
        c = v.shape[1]
        return jax.lax.broadcast_in_dim(v, (N, N, c), (1, 2)).reshape(E, c)

    x = x0
    ea_static = None
    for layer in range(N_LAYERS):
        xi = rep_dst(x)
        xj = rep_src(x)
        cd = xi - xj
        sq = cd * cd
        d2_6 = mm(sq, seg6_ref[...])
        cd = cd / jnp.sqrt(d2_6 + 1e-8)
        if layer == 0:
            ea_static = mm(sq, qd0_ref[...]) + mm(ea4, pea_ref[...])
        edge8 = mm(sq, qd2_ref[...]) + ea_static

        for g in range(INV_SUB):
            mi = layer * 3 + g
            ga = layer * INV_SUB + g
            hr = mm(h, w1r_ref[mi])
            hc = mm(h, w1c_ref[mi])
            et = mm(edge8, w1e_ref[mi])
            m = _silu(rep_dst(hr) + rep_src(hc) + et + b1_ref[mi])
            m = _silu(mm(m, w2_ref[mi]) + b2_ref[mi])
            att2 = jax.nn.sigmoid(mm(m * attw_ref[ga], seg2_ref[...])
                                  + attb_ref[ga])
            ef = m * mm(att2 * em2, sel128_ref[...])
            agg = mm(A, ef) / NORM
            nmlp = _silu(mm(h, n1h_ref[ga]) + mm(agg, n1a_ref[ga])
                         + bn1_ref[ga])
            nmlp = mm(nmlp, n2_ref[ga]) + bn2_ref[ga]
            h = (h + nmlp) * nm128

        mi = layer * 3 + 2
        hr = mm(h, w1r_ref[mi])
        hc = mm(h, w1c_ref[mi])
        et = mm(edge8, w1e_ref[mi])
        it = _silu(rep_dst(hr) + rep_src(hc) + et + b1_ref[mi])
        it = _silu(mm(it, w2_ref[mi]) + b2_ref[mi])
        phi2 = mm(it * c3_ref[layer], seg2_ref[...])
        trans = cd * mm(phi2 * em2, sel6_ref[...])
        aggx = mm(A, trans) / NORM
        x = (x + aggx) * nm6
        h = h * nm128

    hf = (mm(h, outw_ref[...]) + outb_ref[...]) * nm6
    vel = (x - x0) * nm6
    ncnt6 = mm(jnp.sum(nm2, axis=0, keepdims=True), sel6_ref[...])
    mean = jnp.sum(vel, axis=0, keepdims=True) / ncnt6
    vel = vel - mean * nm6

    hf_ref[0] = hf
    vel_ref[0] = vel


def _block_diag2(w):
    s, a, b = w.shape
    z = jnp.zeros((s, 2 * a, 2 * b), jnp.float32)
    return z.at[:, :a, :b].set(w).at[:, a:, b:].set(w)


def kernel(t, xh, node_mask, edge_mask, edge_attributes, params):
    bs, n, dims = xh.shape
    bs2 = bs // 2
    p = params

    w1r, w1c, w1e, b1, w2, b2 = [], [], [], [], [], []
    attw, attb, n1h, n1a, bn1, n2, bn2, c3 = [], [], [], [], [], [], [], []
    for b in range(N_LAYERS):
        for g in range(INV_SUB):
            pre = 'b%d_g%d_' % (b, g)
            w1 = p[pre + 'e1_w']
            w1r.append(w1[:HID]); w1c.append(w1[HID:2 * HID]); w1e.append(w1[2 * HID:])
            b1.append(p[pre + 'e1_b'])
            w2.append(p[pre + 'e2_w']); b2.append(p[pre + 'e2_b'])
            attw.append(p[pre + 'att_w'][:, 0]); attb.append(p[pre + 'att_b'])
            wn1 = p[pre + 'n1_w']
            n1h.append(wn1[:HID]); n1a.append(wn1[HID:])
            bn1.append(p[pre + 'n1_b'])
            n2.append(p[pre + 'n2_w']); bn2.append(p[pre + 'n2_b'])
        pre = 'b%d_eq_' % b
        w1 = p[pre + 'c1_w']
        w1r.append(w1[:HID]); w1c.append(w1[HID:2 * HID]); w1e.append(w1[2 * HID:])
        b1.append(p[pre + 'c1_b'])
        w2.append(p[pre + 'c2_w']); b2.append(p[pre + 'c2_b'])
        c3.append(p[pre + 'c3_w'][:, 0])

    W1R = _block_diag2(jnp.stack(w1r))
    W1C = _block_diag2(jnp.stack(w1c))
    W1E = _block_diag2(jnp.stack(w1e))
    W2 = _block_diag2(jnp.stack(w2))
    N1H = _block_diag2(jnp.stack(n1h))
    N1A = _block_diag2(jnp.stack(n1a))
    N2 = _block_diag2(jnp.stack(n2))
    B1 = jnp.tile(jnp.stack(b1), (1, 2))[:, None, :]
    B2 = jnp.tile(jnp.stack(b2), (1, 2))[:, None, :]
    BN1 = jnp.tile(jnp.stack(bn1), (1, 2))[:, None, :]
    BN2 = jnp.tile(jnp.stack(bn2), (1, 2))[:, None, :]
    ATTW = jnp.tile(jnp.stack(attw), (1, 2))[:, None, :]
    ATTB = jnp.tile(jnp.stack(attb), (1, 2))[:, None, :]
    C3 = jnp.tile(jnp.stack(c3), (1, 2))[:, None, :]
    EMBW = _block_diag2(p['emb_w'][None, :8, :])[0]
    EMBWT = jnp.tile(p['emb_w'][8:9, :], (1, 2))
    EMBB = jnp.tile(p['emb_b'][None, :], (1, 2))
    OUTW = _block_diag2(p['out_w'][None])[0]
    OUTB = jnp.tile(p['out_b'][None, :], (1, 2))

    lane = np.arange(128)
    SEL128 = (lane[None, :] // HID == np.arange(2)[:, None]).astype(np.float32)
    SEL16 = (np.arange(16)[None, :] // 8 == np.arange(2)[:, None]).astype(np.float32)
    SEL6 = (np.arange(6)[None, :] // 3 == np.arange(2)[:, None]).astype(np.float32)
    SEG2 = SEL128.T.copy()
    SEG6 = (np.arange(6)[:, None] // 3 == np.arange(6)[None, :] // 3).astype(np.float32)
    K62 = SEL6.T
    PD2 = np.zeros((2, 8), np.float32); PD2[0, 0] = PD2[1, 4] = 1.0
    PD0 = np.zeros((2, 8), np.float32); PD0[0, 1] = PD0[1, 5] = 1.0
    QD2 = (K62 @ PD2).astype(np.float32)
    QD0 = (K62 @ PD0).astype(np.float32)
    PEA = np.zeros((4, 8), np.float32)
    PEA[0, 2] = PEA[1, 3] = PEA[2, 6] = PEA[3, 7] = 1.0
    AMAT = (np.arange(E)[None, :] // N == np.arange(N)[:, None]).astype(np.float32)

    consts = [jnp.asarray(v) for v in
              (AMAT, SEL128, SEL16, SEL6, SEG2, SEG6, QD2, QD0, PEA)]

    t2 = t.reshape(bs2, 1, 2)
    xh4 = xh.reshape(bs2, 2, n, dims).transpose(0, 2, 1, 3)
    xhh = xh4[:, :, :, :8].reshape(bs2, n, 16)
    xhx = xh4[:, :, :, 8:].reshape(bs2, n, 6)
    nm2 = node_mask.reshape(bs2, 2, n).transpose(0, 2, 1)
    em2 = edge_mask.reshape(bs2, 2, E).transpose(0, 2, 1)
    ea2 = (edge_attributes.reshape(bs2, 2, E, 2)
           .transpose(0, 2, 1, 3).reshape(bs2, E, 4))

    def full(a):
        nd = a.ndim
        return pl.BlockSpec(a.shape, lambda b, _n=nd: (0,) * _n)

    weights = (EMBW, EMBWT, EMBB, W1R, W1C, W1E, B1, W2, B2,
               ATTW, ATTB, N1H, N1A, BN1, N2, BN2, C3, OUTW, OUTB)
    in_specs = [
        pl.BlockSpec((1, 1, 2), lambda b: (b, 0, 0)),
        pl.BlockSpec((1, n, 16), lambda b: (b, 0, 0)),
        pl.BlockSpec((1, n, 6), lambda b: (b, 0, 0)),
        pl.BlockSpec((1, n, 2), lambda b: (b, 0, 0)),
        pl.BlockSpec((1, E, 2), lambda b: (b, 0, 0)),
        pl.BlockSpec((1, E, 4), lambda b: (b, 0, 0)),
    ] + [full(a) for a in weights] + [full(a) for a in consts]
    out_specs = (
        pl.BlockSpec((1, n, 6), lambda b: (b, 0, 0)),
        pl.BlockSpec((1, n, 6), lambda b: (b, 0, 0)),
    )
    out_shape = (
        jax.ShapeDtypeStruct((bs2, n, 6), jnp.float32),
        jax.ShapeDtypeStruct((bs2, n, 6), jnp.float32),
    )

    hf, vel = pl.pallas_call(
        _egnn_kernel,
        grid=(bs2,),
        in_specs=in_specs,
        out_specs=out_specs,
        out_shape=out_shape,
        compiler_params=pltpu.CompilerParams(
            dimension_semantics=("parallel",)),
    )(t2, xhh, xhx, nm2, em2, ea2, *weights, *consts)

    hf = hf.reshape(bs2, n, 2, 3).transpose(0, 2, 1, 3).reshape(bs * n, 3)
    vel = vel.reshape(bs2, n, 2, 3).transpose(0, 2, 1, 3).reshape(bs, n, 3)
    return hf, vel


# speedup vs baseline: 30.9337x; 1.0584x over previous
import numpy as np
import jax
import jax.numpy as jnp
from jax.experimental import pallas as pl
from jax.experimental.pallas import tpu as pltpu

HID = 64
N_LAYERS = 4
INV_SUB = 2
N = 48
E = N * N
G = 2
NN = N * G
EE = E * G
NORM = 100.0


def _silu(v):
    return v * jax.nn.sigmoid(v)


def _egnn_kernel(t_ref, xhh_ref, xhx_ref, nm_ref, em_ref, ea_ref,
                 embw_ref, embwt_ref, embb_ref,
                 w1r_ref, w1c_ref, w1e_ref, b1_ref, w2_ref, b2_ref,
                 attw_ref, attb_ref,
                 n1h_ref, n1a_ref, bn1_ref, n2_ref, bn2_ref,
                 c3_ref, outw_ref, outb_ref,
                 a_ref, sel128_ref, sel16_ref, sel6_ref, seg2_ref,
                 seg6_ref, qd2_ref, qd0_ref, pea_ref, rg_ref, mg_ref,
                 hf_ref, vel_ref):
    nm2 = nm_ref[...].reshape(NN, 2)
    em2 = em_ref[...].reshape(EE, 2)
    ea4 = ea_ref[...].reshape(EE, 4)
    tg = t_ref[0]
    A = a_ref[...]

    def mm(a, b):
        return jnp.dot(a, b, preferred_element_type=jnp.float32)

    nm128 = mm(nm2, sel128_ref[...])
    nm16 = mm(nm2, sel16_ref[...])
    nm6 = mm(nm2, sel6_ref[...])

    h_raw = xhh_ref[...].reshape(NN, 16) * nm16
    x0 = xhx_ref[...].reshape(NN, 6) * nm6

    h = (mm(h_raw, embw_ref[...])
         + mm(rg_ref[...], mm(tg, sel128_ref[...])) * embwt_ref[...]
         + embb_ref[...])

    def rep_dst(v):
        c = v.shape[1]
        return jax.lax.broadcast_in_dim(v, (NN, N, c), (0, 2)).reshape(EE, c)

    def rep_src(v):
        c = v.shape[1]
        parts = [
            jax.lax.broadcast_in_dim(v[p * N:(p + 1) * N], (N, N, c),
                                     (1, 2)).reshape(E, c)
            for p in range(G)
        ]
        return jnp.concatenate(parts, axis=0) if G > 1 else parts[0]

    x = x0
    ea_static = None
    for layer in range(N_LAYERS):
        xi = rep_dst(x)
        xj = rep_src(x)
        cd = xi - xj
        sq = cd * cd
        d2_6 = mm(sq, seg6_ref[...])
        cd = cd / jnp.sqrt(d2_6 + 1e-8)
        if layer == 0:
            ea_static = mm(sq, qd0_ref[...]) + mm(ea4, pea_ref[...])
        edge8 = mm(sq, qd2_ref[...]) + ea_static

        for g in range(INV_SUB):
            mi = layer * 3 + g
            ga = layer * INV_SUB + g
            hr = mm(h, w1r_ref[mi])
            hc = mm(h, w1c_ref[mi])
            et = mm(edge8, w1e_ref[mi])
            m = _silu(rep_dst(hr) + rep_src(hc) + et + b1_ref[mi])
            m = _silu(mm(m, w2_ref[mi]) + b2_ref[mi])
            att2 = jax.nn.sigmoid(mm(m * attw_ref[ga], seg2_ref[...])
                                  + attb_ref[ga])
            ef = m * mm(att2 * em2, sel128_ref[...])
            agg = mm(A, ef) / NORM
            nmlp = _silu(mm(h, n1h_ref[ga]) + mm(agg, n1a_ref[ga])
                         + bn1_ref[ga])
            nmlp = mm(nmlp, n2_ref[ga]) + bn2_ref[ga]
            h = (h + nmlp) * nm128

        mi = layer * 3 + 2
        hr = mm(h, w1r_ref[mi])
        hc = mm(h, w1c_ref[mi])
        et = mm(edge8, w1e_ref[mi])
        it = _silu(rep_dst(hr) + rep_src(hc) + et + b1_ref[mi])
        it = _silu(mm(it, w2_ref[mi]) + b2_ref[mi])
        phi2 = mm(it * c3_ref[layer], seg2_ref[...])
        trans = cd * mm(phi2 * em2, sel6_ref[...])
        aggx = mm(A, trans) / NORM
        x = (x + aggx) * nm6
        h = h * nm128

    hf = (mm(h, outw_ref[...]) + outb_ref[...]) * nm6
    vel = (x - x0) * nm6
    ncnt = mm(mg_ref[...], nm6)
    mean = mm(rg_ref[...], mm(mg_ref[...], vel) / ncnt)
    vel = vel - mean * nm6

    hf_ref[...] = hf.reshape(G, N, 6)
    vel_ref[...] = vel.reshape(G, N, 6)


def _block_diag2(w):
    s, a, b = w.shape
    z = jnp.zeros((s, 2 * a, 2 * b), jnp.float32)
    return z.at[:, :a, :b].set(w).at[:, a:, b:].set(w)


def kernel(t, xh, node_mask, edge_mask, edge_attributes, params):
    bs, n, dims = xh.shape
    bs2 = bs // 2
    p = params

    w1r, w1c, w1e, b1, w2, b2 = [], [], [], [], [], []
    attw, attb, n1h, n1a, bn1, n2, bn2, c3 = [], [], [], [], [], [], [], []
    for b in range(N_LAYERS):
        for g in range(INV_SUB):
            pre = 'b%d_g%d_' % (b, g)
            w1 = p[pre + 'e1_w']
            w1r.append(w1[:HID]); w1c.append(w1[HID:2 * HID]); w1e.append(w1[2 * HID:])
            b1.append(p[pre + 'e1_b'])
            w2.append(p[pre + 'e2_w']); b2.append(p[pre + 'e2_b'])
            attw.append(p[pre + 'att_w'][:, 0]); attb.append(p[pre + 'att_b'])
            wn1 = p[pre + 'n1_w']
            n1h.append(wn1[:HID]); n1a.append(wn1[HID:])
            bn1.append(p[pre + 'n1_b'])
            n2.append(p[pre + 'n2_w']); bn2.append(p[pre + 'n2_b'])
        pre = 'b%d_eq_' % b
        w1 = p[pre + 'c1_w']
        w1r.append(w1[:HID]); w1c.append(w1[HID:2 * HID]); w1e.append(w1[2 * HID:])
        b1.append(p[pre + 'c1_b'])
        w2.append(p[pre + 'c2_w']); b2.append(p[pre + 'c2_b'])
        c3.append(p[pre + 'c3_w'][:, 0])

    W1R = _block_diag2(jnp.stack(w1r))
    W1C = _block_diag2(jnp.stack(w1c))
    W1E = _block_diag2(jnp.stack(w1e))
    W2 = _block_diag2(jnp.stack(w2))
    N1H = _block_diag2(jnp.stack(n1h))
    N1A = _block_diag2(jnp.stack(n1a))
    N2 = _block_diag2(jnp.stack(n2))
    B1 = jnp.tile(jnp.stack(b1), (1, 2))[:, None, :]
    B2 = jnp.tile(jnp.stack(b2), (1, 2))[:, None, :]
    BN1 = jnp.tile(jnp.stack(bn1), (1, 2))[:, None, :]
    BN2 = jnp.tile(jnp.stack(bn2), (1, 2))[:, None, :]
    ATTW = jnp.tile(jnp.stack(attw), (1, 2))[:, None, :]
    ATTB = jnp.tile(jnp.stack(attb), (1, 2))[:, None, :]
    C3 = jnp.tile(jnp.stack(c3), (1, 2))[:, None, :]
    EMBW = _block_diag2(p['emb_w'][None, :8, :])[0]
    EMBWT = jnp.tile(p['emb_w'][8:9, :], (1, 2))
    EMBB = jnp.tile(p['emb_b'][None, :], (1, 2))
    OUTW = _block_diag2(p['out_w'][None])[0]
    OUTB = jnp.tile(p['out_b'][None, :], (1, 2))

    lane = np.arange(128)
    SEL128 = (lane[None, :] // HID == np.arange(2)[:, None]).astype(np.float32)
    SEL16 = (np.arange(16)[None, :] // 8 == np.arange(2)[:, None]).astype(np.float32)
    SEL6 = (np.arange(6)[None, :] // 3 == np.arange(2)[:, None]).astype(np.float32)
    SEG2 = SEL128.T.copy()
    SEG6 = (np.arange(6)[:, None] // 3 == np.arange(6)[None, :] // 3).astype(np.float32)
    K62 = SEL6.T
    PD2 = np.zeros((2, 8), np.float32); PD2[0, 0] = PD2[1, 4] = 1.0
    PD0 = np.zeros((2, 8), np.float32); PD0[0, 1] = PD0[1, 5] = 1.0
    QD2 = (K62 @ PD2).astype(np.float32)
    QD0 = (K62 @ PD0).astype(np.float32)
    PEA = np.zeros((4, 8), np.float32)
    PEA[0, 2] = PEA[1, 3] = PEA[2, 6] = PEA[3, 7] = 1.0
    AMAT = (np.arange(EE)[None, :] // N == np.arange(NN)[:, None]).astype(np.float32)
    RG = (np.arange(NN)[:, None] // N == np.arange(G)[None, :]).astype(np.float32)
    MG = RG.T.copy()

    consts = [jnp.asarray(v) for v in
              (AMAT, SEL128, SEL16, SEL6, SEG2, SEG6, QD2, QD0, PEA, RG, MG)]

    ng = bs2 // G
    t2 = t.reshape(ng, G, 2)
    xh4 = xh.reshape(bs2, 2, n, dims).transpose(0, 2, 1, 3)
    xhh = xh4[:, :, :, :8].reshape(bs2, n, 16)
    xhx = xh4[:, :, :, 8:].reshape(bs2, n, 6)
    nm2 = node_mask.reshape(bs2, 2, n).transpose(0, 2, 1)
    em2 = edge_mask.reshape(bs2, 2, E).transpose(0, 2, 1)
    ea2 = (edge_attributes.reshape(bs2, 2, E, 2)
           .transpose(0, 2, 1, 3).reshape(bs2, E, 4))

    def full(a):
        nd = a.ndim
        return pl.BlockSpec(a.shape, lambda b, _n=nd: (0,) * _n)

    weights = (EMBW, EMBWT, EMBB, W1R, W1C, W1E, B1, W2, B2,
               ATTW, ATTB, N1H, N1A, BN1, N2, BN2, C3, OUTW, OUTB)
    in_specs = [
        pl.BlockSpec((1, G, 2), lambda b: (b, 0, 0)),
        pl.BlockSpec((G, n, 16), lambda b: (b, 0, 0)),
        pl.BlockSpec((G, n, 6), lambda b: (b, 0, 0)),
        pl.BlockSpec((G, n, 2), lambda b: (b, 0, 0)),
        pl.BlockSpec((G, E, 2), lambda b: (b, 0, 0)),
        pl.BlockSpec((G, E, 4), lambda b: (b, 0, 0)),
    ] + [full(a) for a in weights] + [full(a) for a in consts]
    out_specs = (
        pl.BlockSpec((G, n, 6), lambda b: (b, 0, 0)),
        pl.BlockSpec((G, n, 6), lambda b: (b, 0, 0)),
    )
    out_shape = (
        jax.ShapeDtypeStruct((bs2, n, 6), jnp.float32),
        jax.ShapeDtypeStruct((bs2, n, 6), jnp.float32),
    )

    hf, vel = pl.pallas_call(
        _egnn_kernel,
        grid=(ng,),
        in_specs=in_specs,
        out_specs=out_specs,
        out_shape=out_shape,
        compiler_params=pltpu.CompilerParams(
            dimension_semantics=("parallel",)),
    )(t2, xhh, xhx, nm2, em2, ea2, *weights, *consts)

    hf = hf.reshape(bs2, n, 2, 3).transpose(0, 2, 1, 3).reshape(bs * n, 3)
    vel = vel.reshape(bs2, n, 2, 3).transpose(0, 2, 1, 3).reshape(bs, n, 3)
    return hf, vel
